# Initial kernel scaffold; baseline (speedup 1.0000x reference)
import jax
import jax.numpy as jnp
from jax.experimental import pallas as pl


def kernel(indices, table, W1, b1, W2, b2, W3, b3):
    raise NotImplementedError("write your pallas kernel here")


# speedup vs baseline: 9.0894x; 9.0894x over previous
import functools

import jax
import jax.numpy as jnp
from jax import lax
from jax.experimental import pallas as pl
from jax.experimental.pallas import tpu as pltpu
from jax.experimental.pallas import tpu_sc as plsc

B, L = 4096, 50
V, D = 12 * 64 * 64, 256
H = 32
HP = 128
NC, NS = 2, 16
NW = NC * NS
BPW = B // NW
_ACC_UNROLL = 8


def _proj_body(tbl_ref, w1t_ref, out_ref):
    out_ref[...] = jnp.dot(tbl_ref[...], w1t_ref[...],
                           preferred_element_type=jnp.float32)


def _project_table(table, w1t):
    blk = 4096
    return pl.pallas_call(
        _proj_body,
        grid=(V // blk,),
        in_specs=[
            pl.BlockSpec((blk, D), lambda i: (i, 0)),
            pl.BlockSpec((D, HP), lambda i: (0, 0)),
        ],
        out_specs=pl.BlockSpec((blk, HP), lambda i: (i, 0)),
        out_shape=jax.ShapeDtypeStruct((V, HP), jnp.float32),
    )(table, w1t)


def _sc_body(t1, idxw, x1, idx_v, buf0, buf1, acc, sem0, sem1):
    wid = lax.axis_index("s") * NC + lax.axis_index("c")
    pltpu.sync_copy(idxw.at[wid], idx_v)

    def _gather(l, buf, sem):
        pltpu.async_copy(t1.at[idx_v.at[l]], buf, sem)

    def _wait(buf, sem):
        pltpu.make_async_copy(t1.at[idx_v.at[0]], buf, sem).wait()

    def _accum(buf, first):
        def body(j, c):
            b = j * _ACC_UNROLL
            for u in range(_ACC_UNROLL):
                for h in range(2):
                    v = buf[b + u, pl.ds(16 * h, 16)]
                    if first:
                        acc[b + u, pl.ds(16 * h, 16)] = v
                    else:
                        plsc.addupdate(acc.at[b + u, pl.ds(16 * h, 16)], v)
            return c
        lax.fori_loop(0, BPW // _ACC_UNROLL, body, 0)

    _gather(0, buf0, sem0)
    _wait(buf0, sem0)
    _gather(1, buf1, sem1)
    _accum(buf0, True)
    _wait(buf1, sem1)
    _gather(2, buf0, sem0)
    _accum(buf1, False)

    def loop_body(i, c):
        l2 = 2 * i
        _wait(buf0, sem0)
        _gather(l2 + 1, buf1, sem1)
        _accum(buf0, False)
        _wait(buf1, sem1)

        @pl.when(i < (L // 2) - 1)
        def _():
            _gather(l2 + 2, buf0, sem0)

        _accum(buf1, False)
        return c

    lax.fori_loop(1, L // 2, loop_body, 0)
    pltpu.sync_copy(acc, x1.at[pl.ds(wid * BPW, BPW)])


_sc_gather_sum = functools.partial(
    pl.kernel,
    out_type=jax.ShapeDtypeStruct((B, H), jnp.float32),
    mesh=plsc.VectorSubcoreMesh(core_axis_name="c", subcore_axis_name="s"),
    scratch_types=[
        pltpu.VMEM((L, BPW), jnp.int32),
        pltpu.VMEM((BPW, HP), jnp.float32),
        pltpu.VMEM((BPW, HP), jnp.float32),
        pltpu.VMEM((BPW, H), jnp.float32),
        pltpu.SemaphoreType.DMA,
        pltpu.SemaphoreType.DMA,
    ],
)(_sc_body)


def _mlp_body(x_ref, b1_ref, w2t_ref, b2_ref, w3t_ref, b3_ref, out_ref):
    h1 = jnp.maximum(x_ref[...] + b1_ref[...], 0.0)
    h2 = jnp.dot(h1, w2t_ref[...], preferred_element_type=jnp.float32)
    h2 = jnp.maximum(h2 + b2_ref[...], 0.0)
    out_ref[...] = (jnp.dot(h2, w3t_ref[...], preferred_element_type=jnp.float32)
                    + b3_ref[...])


def _mlp(x1, b1, w2t, b2, w3t, b3):
    return pl.pallas_call(
        _mlp_body,
        out_shape=jax.ShapeDtypeStruct((B, 1), jnp.float32),
    )(x1, b1.reshape(1, H), w2t, b2.reshape(1, H), w3t, b3.reshape(1, 1))


def kernel(indices, table, W1, b1, W2, b2, W3, b3):
    w1t = jnp.pad(W1.T, ((0, 0), (0, HP - H)))
    t1 = _project_table(table, w1t)
    idxw = indices.astype(jnp.int32).reshape(NW, BPW, L).transpose(0, 2, 1)
    x1 = _sc_gather_sum(t1, idxw)
    out = _mlp(x1, b1, W2.T, b2, W3.T, b3)
    return out[:, 0]
